# baseline (speedup 1.0000x reference)
import functools

import jax
import jax.numpy as jnp
from jax import lax
from jax.experimental import pallas as pl
from jax.experimental.pallas import tpu as pltpu
from jax.experimental.pallas import tpu_sc as plsc

Q = 320000
D = 128
C = 10
CP = 16
NC = 2
NS = 16
NW = NC * NS
T = 128000
B = 2000
NB = T // B
SC_ROWS = Q - T
ROWS_PER_W = SC_ROWS // NW
P = 500
NP = ROWS_PER_W // P
NEG_INF = float("-inf")


def _sc_segmax_body(
    q_hbm, starts_hbm, ends_hbm, out_hbm, buf0, buf1, acc, sv, ev, sem0, sem1
):
    wid = lax.axis_index("s") * NC + lax.axis_index("c")
    lo = T + wid * ROWS_PER_W

    pltpu.sync_copy(starts_hbm, sv)
    pltpu.sync_copy(ends_hbm, ev)

    neg = jnp.full((16,), NEG_INF, jnp.float32)

    def init_body(i, _):
        acc[pl.ds(i * 16, 16)] = neg
        return 0

    lax.fori_loop(0, (CP * D) // 16, init_body, 0)

    svv = sv[pl.ds(0, 16)]
    evv = ev[pl.ds(0, 16)]
    starts_s = [svv[c] for c in range(C)]
    ends_s = [evv[c] for c in range(C)]

    def start(p, buf, sem):
        pltpu.async_copy(q_hbm.at[pl.ds((lo + p * P) * D, P * D)], buf, sem)

    def wait(buf, sem):
        pltpu.make_async_copy(q_hbm.at[pl.ds(0, P * D)], buf, sem).wait()

    def process(p, buf):
        row0 = lo + p * P
        for c in range(C):
            rs = jnp.clip(starts_s[c] - row0, 0, P)
            re = jnp.clip(ends_s[c] - row0, 0, P)
            a = tuple(acc[pl.ds(c * D + dc * 16, 16)] for dc in range(8))

            def rbody(r, a):
                base = r * D
                return tuple(
                    jnp.maximum(a[dc], buf[pl.ds(base + dc * 16, 16)])
                    for dc in range(8)
                )

            a = lax.fori_loop(rs, re, rbody, a)
            for dc in range(8):
                acc[pl.ds(c * D + dc * 16, 16)] = a[dc]

    start(0, buf0, sem0)
    start(1, buf1, sem1)

    def gbody(g, _):
        p0 = 2 * g
        wait(buf0, sem0)
        process(p0, buf0)

        @pl.when(p0 + 2 < NP)
        def _():
            start(p0 + 2, buf0, sem0)

        wait(buf1, sem1)
        process(p0 + 1, buf1)

        @pl.when(p0 + 3 < NP)
        def _():
            start(p0 + 3, buf1, sem1)

        return 0

    lax.fori_loop(0, NP // 2, gbody, 0)
    pltpu.sync_copy(acc, out_hbm.at[pl.ds(wid * CP * D, CP * D)])


_sc_segmax = functools.partial(
    pl.kernel,
    out_type=jax.ShapeDtypeStruct((NW * CP * D,), jnp.float32),
    mesh=plsc.VectorSubcoreMesh(core_axis_name="c", subcore_axis_name="s"),
    scratch_types=[
        pltpu.VMEM((P * D,), jnp.float32),
        pltpu.VMEM((P * D,), jnp.float32),
        pltpu.VMEM((CP * D,), jnp.float32),
        pltpu.VMEM((16,), jnp.int32),
        pltpu.VMEM((16,), jnp.int32),
        pltpu.SemaphoreType.DMA,
        pltpu.SemaphoreType.DMA,
    ],
)(_sc_segmax_body)


def _tc_segmax_body(sv_ref, ev_ref, x_ref, out_ref):
    i = pl.program_id(0)
    base = i * B

    @pl.when(i == 0)
    def _():
        out_ref[:] = jnp.full((CP, D), NEG_INF, jnp.float32)

    x = x_ref[:]
    first = jnp.int32(0)
    last = jnp.int32(0)
    for c in range(C):
        first = first + (ev_ref[c] <= base).astype(jnp.int32)
        last = last + (ev_ref[c] <= base + B - 1).astype(jnp.int32)

    @pl.when(first == last)
    def _():
        m = jnp.max(x, axis=0, keepdims=True)
        cur = out_ref[pl.ds(first, 1), :]
        out_ref[pl.ds(first, 1), :] = jnp.maximum(cur, m)

    @pl.when(first != last)
    def _():
        rowid = base + lax.broadcasted_iota(jnp.int32, (B, 1), 0)
        for c in range(C):
            mask = (rowid >= sv_ref[c]) & (rowid < ev_ref[c])
            m = jnp.max(jnp.where(mask, x, NEG_INF), axis=0, keepdims=True)
            cur = out_ref[pl.ds(c, 1), :]
            out_ref[pl.ds(c, 1), :] = jnp.maximum(cur, m)


_tc_segmax = pl.pallas_call(
    _tc_segmax_body,
    grid_spec=pltpu.PrefetchScalarGridSpec(
        num_scalar_prefetch=2,
        grid=(NB,),
        in_specs=[pl.BlockSpec((B, D), lambda i, sv, ev: (i, 0))],
        out_specs=pl.BlockSpec((CP, D), lambda i, sv, ev: (0, 0)),
    ),
    out_shape=jax.ShapeDtypeStruct((CP, D), jnp.float32),
)


def _tc_gcn_body(
    part_ref, tcp_ref, counts_ref, pad_ref, w_ref, b_ref, cc_ref, out_ref
):
    seg = tcp_ref[:]
    for i in range(NW):
        seg = jnp.maximum(seg, part_ref[i])
    has = counts_ref[:] > 0
    core = jnp.where(has, seg, pad_ref[:])

    cc = cc_ref[:]
    rr = lax.broadcasted_iota(jnp.int32, (CP, CP), 0)
    cidx = lax.broadcasted_iota(jnp.int32, (CP, CP), 1)
    eye = jnp.where(rr == cidx, 1.0, 0.0).astype(jnp.float32)
    a_hat = cc + eye
    deg = jnp.sum(a_hat, axis=1, keepdims=True)
    dinv = jnp.where(deg > 0, lax.rsqrt(deg), 0.0)
    t = jnp.dot(a_hat, core * dinv, preferred_element_type=jnp.float32)
    h = jnp.dot(t * dinv, w_ref[:], preferred_element_type=jnp.float32)
    out_ref[:] = jnp.maximum(h + b_ref[:], 0.0)


_tc_gcn = pl.pallas_call(
    _tc_gcn_body,
    out_shape=jax.ShapeDtypeStruct((CP, D), jnp.float32),
)


def kernel(prev_assign, q_embeddings, padding_emb, W, b, core_con):
    pa = prev_assign.astype(jnp.int32)
    starts = jnp.sum(
        (pa[:, None] < jnp.arange(C + 1, dtype=jnp.int32)[None, :]).astype(jnp.int32),
        axis=0,
    )
    pad6 = jnp.full((CP - C,), Q, jnp.int32)
    starts16 = jnp.concatenate([starts[:C], pad6])
    ends16 = jnp.concatenate([starts[1:], pad6])
    counts16 = jnp.concatenate(
        [starts[1:] - starts[:C], jnp.zeros((CP - C,), jnp.int32)]
    ).reshape(CP, 1)

    partials = _sc_segmax(q_embeddings.reshape(-1), starts16, ends16)
    partials = partials.reshape(NW, CP, D)
    tc_part = _tc_segmax(starts16, ends16, q_embeddings)

    cc16 = jnp.zeros((CP, CP), jnp.float32).at[:C, :C].set(core_con)
    out16 = _tc_gcn(
        partials,
        tc_part,
        counts16,
        padding_emb.reshape(1, D),
        W,
        b.reshape(1, D),
        cc16,
    )
    return out16[:C]


# speedup vs baseline: 1.0699x; 1.0699x over previous
import functools

import jax
import jax.numpy as jnp
from jax import lax
from jax.experimental import pallas as pl
from jax.experimental.pallas import tpu as pltpu
from jax.experimental.pallas import tpu_sc as plsc

Q = 320000
D = 128
C = 10
CP = 16
NC = 2
NS = 16
NW = NC * NS
ROWS_PER_W = Q // NW
P = 400
NP = ROWS_PER_W // P
NSEARCH = 14
NEG_INF = float("-inf")


def _sc_segmax_body(q_hbm, pa_hbm, out_hbm, buf0, buf1, acc, pa_buf, bounds_smem, sem0, sem1):
    wid = lax.axis_index("s") * NC + lax.axis_index("c")
    lo = wid * ROWS_PER_W

    def start(p, buf, sem):
        pltpu.async_copy(q_hbm.at[pl.ds((lo + p * P) * D, P * D)], buf, sem)

    def wait(buf, sem):
        pltpu.make_async_copy(q_hbm.at[pl.ds(0, P * D)], buf, sem).wait()

    start(0, buf0, sem0)
    start(1, buf1, sem1)

    pltpu.sync_copy(pa_hbm.at[pl.ds(lo, ROWS_PER_W)], pa_buf.at[pl.ds(0, ROWS_PER_W)])

    def cbody(c, _):
        def sbody(_, lh):
            lo_i, hi_i = lh
            mid = (lo_i + hi_i) // 2
            v = pa_buf[pl.ds(mid, 16)]
            pred = v[0] < c
            return (
                jnp.where(pred, mid + 1, lo_i),
                jnp.where(pred, hi_i, mid),
            )

        l, _ = lax.fori_loop(
            0, NSEARCH, sbody, (jnp.int32(0), jnp.int32(ROWS_PER_W))
        )
        bounds_smem[c] = l
        return 0

    bounds_smem[0] = jnp.int32(0)
    bounds_smem[C] = jnp.int32(ROWS_PER_W)
    lax.fori_loop(1, C, cbody, 0)
    starts_s = [bounds_smem[c] for c in range(C + 1)]

    neg = jnp.full((16,), NEG_INF, jnp.float32)

    def init_body(i, _):
        acc[pl.ds(i * 16, 16)] = neg
        return 0

    lax.fori_loop(0, (CP * D) // 16, init_body, 0)

    def process(p, buf):
        row0 = p * P
        for c in range(C):
            rs = jnp.clip(starts_s[c] - row0, 0, P)
            re = jnp.clip(starts_s[c + 1] - row0, 0, P)
            a = tuple(acc[pl.ds(c * D + dc * 16, 16)] for dc in range(8))

            def rbody(r, a):
                base = r * D
                return tuple(
                    jnp.maximum(a[dc], buf[pl.ds(base + dc * 16, 16)])
                    for dc in range(8)
                )

            a = lax.fori_loop(rs, re, rbody, a)
            for dc in range(8):
                acc[pl.ds(c * D + dc * 16, 16)] = a[dc]

    def gbody(g, _):
        p0 = 2 * g
        wait(buf0, sem0)
        process(p0, buf0)

        @pl.when(p0 + 2 < NP)
        def _():
            start(p0 + 2, buf0, sem0)

        wait(buf1, sem1)
        process(p0 + 1, buf1)

        @pl.when(p0 + 3 < NP)
        def _():
            start(p0 + 3, buf1, sem1)

        return 0

    lax.fori_loop(0, NP // 2, gbody, 0)
    if NP % 2:
        wait(buf0, sem0)
        process(NP - 1, buf0)
    pltpu.sync_copy(acc, out_hbm.at[pl.ds(wid * CP * D, CP * D)])


_sc_segmax = functools.partial(
    pl.kernel,
    out_type=jax.ShapeDtypeStruct((NW * CP * D,), jnp.float32),
    mesh=plsc.VectorSubcoreMesh(core_axis_name="c", subcore_axis_name="s"),
    scratch_types=[
        pltpu.VMEM((P * D,), jnp.float32),
        pltpu.VMEM((P * D,), jnp.float32),
        pltpu.VMEM((CP * D,), jnp.float32),
        pltpu.VMEM((ROWS_PER_W + 16,), jnp.int32),
        pltpu.SMEM((16,), jnp.int32),
        pltpu.SemaphoreType.DMA,
        pltpu.SemaphoreType.DMA,
    ],
)(_sc_segmax_body)


def _tc_gcn_body(part_ref, pad_ref, w_ref, b_ref, cc_ref, out_ref):
    seg = part_ref[0]
    for i in range(1, NW):
        seg = jnp.maximum(seg, part_ref[i])
    core = jnp.where(seg > NEG_INF, seg, pad_ref[:])

    cc = cc_ref[:]
    rr = lax.broadcasted_iota(jnp.int32, (CP, CP), 0)
    cidx = lax.broadcasted_iota(jnp.int32, (CP, CP), 1)
    eye = jnp.where(rr == cidx, 1.0, 0.0).astype(jnp.float32)
    a_hat = cc + eye
    deg = jnp.sum(a_hat, axis=1, keepdims=True)
    dinv = jnp.where(deg > 0, lax.rsqrt(deg), 0.0)
    t = jnp.dot(a_hat, core * dinv, preferred_element_type=jnp.float32)
    h = jnp.dot(t * dinv, w_ref[:], preferred_element_type=jnp.float32)
    out_ref[:] = jnp.maximum(h + b_ref[:], 0.0)


_tc_gcn = pl.pallas_call(
    _tc_gcn_body,
    out_shape=jax.ShapeDtypeStruct((CP, D), jnp.float32),
)


def kernel(prev_assign, q_embeddings, padding_emb, W, b, core_con):
    pa = prev_assign.astype(jnp.int32)
    partials = _sc_segmax(q_embeddings.reshape(-1), pa)
    partials = partials.reshape(NW, CP, D)

    cc16 = jnp.zeros((CP, CP), jnp.float32).at[:C, :C].set(core_con)
    out16 = _tc_gcn(
        partials,
        padding_emb.reshape(1, D),
        W,
        b.reshape(1, D),
        cc16,
    )
    return out16[:C]


# speedup vs baseline: 1.0701x; 1.0003x over previous
import functools

import jax
import jax.numpy as jnp
from jax import lax
from jax.experimental import pallas as pl
from jax.experimental.pallas import tpu as pltpu
from jax.experimental.pallas import tpu_sc as plsc

Q = 320000
D = 128
C = 10
CP = 16
NC = 2
NS = 16
NW = NC * NS
ROWS_PER_W = Q // NW
P = 400
NP = ROWS_PER_W // P
NSEARCH = 14
NEG_INF = float("-inf")


def _sc_segmax_body(q_hbm, pa_hbm, out_hbm, buf0, buf1, acc, pa_buf, bounds_smem, sem0, sem1):
    wid = lax.axis_index("s") * NC + lax.axis_index("c")
    lo = wid * ROWS_PER_W

    def start(p, buf, sem):
        pltpu.async_copy(q_hbm.at[pl.ds((lo + p * P) * D, P * D)], buf, sem)

    def wait(buf, sem):
        pltpu.make_async_copy(q_hbm.at[pl.ds(0, P * D)], buf, sem).wait()

    start(0, buf0, sem0)
    start(1, buf1, sem1)

    pltpu.sync_copy(pa_hbm.at[pl.ds(lo, ROWS_PER_W)], pa_buf.at[pl.ds(0, ROWS_PER_W)])

    def cbody(c, _):
        def sbody(_, lh):
            lo_i, hi_i = lh
            mid = (lo_i + hi_i) // 2
            v = pa_buf[pl.ds(mid, 16)]
            pred = v[0] < c
            return (
                jnp.where(pred, mid + 1, lo_i),
                jnp.where(pred, hi_i, mid),
            )

        l, _ = lax.fori_loop(
            0, NSEARCH, sbody, (jnp.int32(0), jnp.int32(ROWS_PER_W))
        )
        bounds_smem[c] = l
        return 0

    bounds_smem[0] = jnp.int32(0)
    bounds_smem[C] = jnp.int32(ROWS_PER_W)
    lax.fori_loop(1, C, cbody, 0)
    starts_s = [bounds_smem[c] for c in range(C + 1)]

    neg = jnp.full((16,), NEG_INF, jnp.float32)

    def init_body(i, _):
        acc[pl.ds(i * 16, 16)] = neg
        return 0

    lax.fori_loop(0, (CP * D) // 16, init_body, 0)

    def process(p, buf):
        row0 = p * P
        for c in range(C):
            rs = jnp.clip(starts_s[c] - row0, 0, P)
            re = jnp.clip(starts_s[c + 1] - row0, 0, P)
            a = tuple(acc[pl.ds(c * D + dc * 16, 16)] for dc in range(8))

            def rbody(r, a):
                base = r * D
                return tuple(
                    jnp.maximum(a[dc], buf[pl.ds(base + dc * 16, 16)])
                    for dc in range(8)
                )

            a = plsc.parallel_loop(rs, re, unroll=4, carry=a)(rbody)
            for dc in range(8):
                acc[pl.ds(c * D + dc * 16, 16)] = a[dc]

    def gbody(g, _):
        p0 = 2 * g
        wait(buf0, sem0)
        process(p0, buf0)

        @pl.when(p0 + 2 < NP)
        def _():
            start(p0 + 2, buf0, sem0)

        wait(buf1, sem1)
        process(p0 + 1, buf1)

        @pl.when(p0 + 3 < NP)
        def _():
            start(p0 + 3, buf1, sem1)

        return 0

    lax.fori_loop(0, NP // 2, gbody, 0)
    if NP % 2:
        wait(buf0, sem0)
        process(NP - 1, buf0)
    pltpu.sync_copy(acc, out_hbm.at[pl.ds(wid * CP * D, CP * D)])


_sc_segmax = functools.partial(
    pl.kernel,
    out_type=jax.ShapeDtypeStruct((NW * CP * D,), jnp.float32),
    mesh=plsc.VectorSubcoreMesh(core_axis_name="c", subcore_axis_name="s"),
    scratch_types=[
        pltpu.VMEM((P * D,), jnp.float32),
        pltpu.VMEM((P * D,), jnp.float32),
        pltpu.VMEM((CP * D,), jnp.float32),
        pltpu.VMEM((ROWS_PER_W + 16,), jnp.int32),
        pltpu.SMEM((16,), jnp.int32),
        pltpu.SemaphoreType.DMA,
        pltpu.SemaphoreType.DMA,
    ],
)(_sc_segmax_body)


def _tc_gcn_body(part_ref, pad_ref, w_ref, b_ref, cc_ref, out_ref):
    seg = part_ref[0]
    for i in range(1, NW):
        seg = jnp.maximum(seg, part_ref[i])
    core = jnp.where(seg > NEG_INF, seg, pad_ref[:])

    cc = cc_ref[:]
    rr = lax.broadcasted_iota(jnp.int32, (CP, CP), 0)
    cidx = lax.broadcasted_iota(jnp.int32, (CP, CP), 1)
    eye = jnp.where(rr == cidx, 1.0, 0.0).astype(jnp.float32)
    a_hat = cc + eye
    deg = jnp.sum(a_hat, axis=1, keepdims=True)
    dinv = jnp.where(deg > 0, lax.rsqrt(deg), 0.0)
    t = jnp.dot(a_hat, core * dinv, preferred_element_type=jnp.float32)
    h = jnp.dot(t * dinv, w_ref[:], preferred_element_type=jnp.float32)
    out_ref[:] = jnp.maximum(h + b_ref[:], 0.0)


_tc_gcn = pl.pallas_call(
    _tc_gcn_body,
    out_shape=jax.ShapeDtypeStruct((CP, D), jnp.float32),
)


def kernel(prev_assign, q_embeddings, padding_emb, W, b, core_con):
    pa = prev_assign.astype(jnp.int32)
    partials = _sc_segmax(q_embeddings.reshape(-1), pa)
    partials = partials.reshape(NW, CP, D)

    cc16 = jnp.zeros((CP, CP), jnp.float32).at[:C, :C].set(core_con)
    out16 = _tc_gcn(
        partials,
        padding_emb.reshape(1, D),
        W,
        b.reshape(1, D),
        cc16,
    )
    return out16[:C]


# speedup vs baseline: 1.0890x; 1.0177x over previous
import functools

import jax
import jax.numpy as jnp
from jax import lax
from jax.experimental import pallas as pl
from jax.experimental.pallas import tpu as pltpu
from jax.experimental.pallas import tpu_sc as plsc

Q = 320000
D = 128
C = 10
CP = 16
NC = 2
NS = 16
NW = NC * NS
BT = 1024
T = 86 * BT
NBT = T // BT
SC_ROWS = Q - T
ROWS_PER_W = SC_ROWS // NW
P = 302
NP = ROWS_PER_W // P
NSEARCH = 14
NEG_INF = float("-inf")


def _sc_segmax_body(q_hbm, pa_hbm, out_hbm, buf0, buf1, acc, pa_buf, bounds_smem, sem0, sem1):
    wid = lax.axis_index("s") * NC + lax.axis_index("c")
    lo = T + wid * ROWS_PER_W

    def start(p, buf, sem):
        pltpu.async_copy(q_hbm.at[pl.ds((lo + p * P) * D, P * D)], buf, sem)

    def wait(buf, sem):
        pltpu.make_async_copy(q_hbm.at[pl.ds(0, P * D)], buf, sem).wait()

    start(0, buf0, sem0)
    start(1, buf1, sem1)

    pltpu.sync_copy(pa_hbm.at[pl.ds(lo, ROWS_PER_W)], pa_buf.at[pl.ds(0, ROWS_PER_W)])

    def cbody(c, _):
        def sbody(_, lh):
            lo_i, hi_i = lh
            mid = (lo_i + hi_i) // 2
            v = pa_buf[pl.ds(mid, 16)]
            pred = v[0] < c
            return (
                jnp.where(pred, mid + 1, lo_i),
                jnp.where(pred, hi_i, mid),
            )

        l, _ = lax.fori_loop(
            0, NSEARCH, sbody, (jnp.int32(0), jnp.int32(ROWS_PER_W))
        )
        bounds_smem[c] = l
        return 0

    bounds_smem[0] = jnp.int32(0)
    bounds_smem[C] = jnp.int32(ROWS_PER_W)
    lax.fori_loop(1, C, cbody, 0)
    starts_s = [bounds_smem[c] for c in range(C + 1)]

    neg = jnp.full((16,), NEG_INF, jnp.float32)

    def init_body(i, _):
        acc[pl.ds(i * 16, 16)] = neg
        return 0

    lax.fori_loop(0, (CP * D) // 16, init_body, 0)

    def process(p, buf):
        row0 = p * P
        for c in range(C):
            rs = jnp.clip(starts_s[c] - row0, 0, P)
            re = jnp.clip(starts_s[c + 1] - row0, 0, P)
            a = tuple(acc[pl.ds(c * D + dc * 16, 16)] for dc in range(8))

            def rbody(r, a):
                base = r * D
                return tuple(
                    jnp.maximum(a[dc], buf[pl.ds(base + dc * 16, 16)])
                    for dc in range(8)
                )

            a = plsc.parallel_loop(rs, re, unroll=4, carry=a)(rbody)
            for dc in range(8):
                acc[pl.ds(c * D + dc * 16, 16)] = a[dc]

    def gbody(g, _):
        p0 = 2 * g
        wait(buf0, sem0)
        process(p0, buf0)

        @pl.when(p0 + 2 < NP)
        def _():
            start(p0 + 2, buf0, sem0)

        wait(buf1, sem1)
        process(p0 + 1, buf1)

        @pl.when(p0 + 3 < NP)
        def _():
            start(p0 + 3, buf1, sem1)

        return 0

    lax.fori_loop(0, NP // 2, gbody, 0)
    if NP % 2:
        wait(buf0, sem0)
        process(NP - 1, buf0)
    pltpu.sync_copy(acc, out_hbm.at[pl.ds(wid * CP * D, CP * D)])


_sc_segmax = functools.partial(
    pl.kernel,
    out_type=jax.ShapeDtypeStruct((NW * CP * D,), jnp.float32),
    mesh=plsc.VectorSubcoreMesh(core_axis_name="c", subcore_axis_name="s"),
    scratch_types=[
        pltpu.VMEM((P * D,), jnp.float32),
        pltpu.VMEM((P * D,), jnp.float32),
        pltpu.VMEM((CP * D,), jnp.float32),
        pltpu.VMEM((ROWS_PER_W + 16,), jnp.int32),
        pltpu.SMEM((16,), jnp.int32),
        pltpu.SemaphoreType.DMA,
        pltpu.SemaphoreType.DMA,
    ],
)(_sc_segmax_body)


def _tc_segmax_body(pa_ref, x_ref, out_ref):
    i = pl.program_id(0)
    base_rows = BT // 128

    @pl.when(i == 0)
    def _():
        out_ref[:] = jnp.full((CP, D), NEG_INF, jnp.float32)

    pa_blk = pa_ref[:]
    x = x_ref[:]
    first = pa_blk[0, 0]
    last = pa_blk[base_rows - 1, 127]

    @pl.when(first == last)
    def _():
        m = jnp.max(x, axis=0, keepdims=True)
        cur = out_ref[pl.ds(first, 1), :]
        out_ref[pl.ds(first, 1), :] = jnp.maximum(cur, m)

    @pl.when(first != last)
    def _():
        riota = lax.broadcasted_iota(jnp.int32, (BT, 1), 0)
        less = [jnp.int32(0)]
        for c in range(C):
            less.append(jnp.sum((pa_blk < (c + 1)).astype(jnp.int32)))
        for c in range(C):
            mask = (riota >= less[c]) & (riota < less[c + 1])
            m = jnp.max(jnp.where(mask, x, NEG_INF), axis=0, keepdims=True)
            cur = out_ref[pl.ds(c, 1), :]
            out_ref[pl.ds(c, 1), :] = jnp.maximum(cur, m)


_tc_segmax = pl.pallas_call(
    _tc_segmax_body,
    grid=(NBT,),
    in_specs=[
        pl.BlockSpec((BT // 128, 128), lambda i: (i, 0)),
        pl.BlockSpec((BT, D), lambda i: (i, 0)),
    ],
    out_specs=pl.BlockSpec((CP, D), lambda i: (0, 0)),
    out_shape=jax.ShapeDtypeStruct((CP, D), jnp.float32),
)


def _tc_gcn_body(part_ref, tcp_ref, pad_ref, w_ref, b_ref, cc_ref, out_ref):
    seg = tcp_ref[:]
    for i in range(NW):
        seg = jnp.maximum(seg, part_ref[i])
    core = jnp.where(seg > NEG_INF, seg, pad_ref[:])

    cc = cc_ref[:]
    rr = lax.broadcasted_iota(jnp.int32, (CP, CP), 0)
    cidx = lax.broadcasted_iota(jnp.int32, (CP, CP), 1)
    eye = jnp.where(rr == cidx, 1.0, 0.0).astype(jnp.float32)
    a_hat = cc + eye
    deg = jnp.sum(a_hat, axis=1, keepdims=True)
    dinv = jnp.where(deg > 0, lax.rsqrt(deg), 0.0)
    t = jnp.dot(a_hat, core * dinv, preferred_element_type=jnp.float32)
    h = jnp.dot(t * dinv, w_ref[:], preferred_element_type=jnp.float32)
    out_ref[:] = jnp.maximum(h + b_ref[:], 0.0)


_tc_gcn = pl.pallas_call(
    _tc_gcn_body,
    out_shape=jax.ShapeDtypeStruct((CP, D), jnp.float32),
)


def kernel(prev_assign, q_embeddings, padding_emb, W, b, core_con):
    pa = prev_assign.astype(jnp.int32)
    partials = _sc_segmax(q_embeddings.reshape(-1), pa)
    partials = partials.reshape(NW, CP, D)
    tc_part = _tc_segmax(pa.reshape(Q // 128, 128), q_embeddings)

    cc16 = jnp.zeros((CP, CP), jnp.float32).at[:C, :C].set(core_con)
    out16 = _tc_gcn(
        partials,
        tc_part,
        padding_emb.reshape(1, D),
        W,
        b.reshape(1, D),
        cc16,
    )
    return out16[:C]


# speedup vs baseline: 1.1753x; 1.0792x over previous
import functools

import jax
import jax.numpy as jnp
from jax import lax
from jax.experimental import pallas as pl
from jax.experimental.pallas import tpu as pltpu
from jax.experimental.pallas import tpu_sc as plsc

Q = 320000
D = 128
C = 10
CP = 16
NC = 2
NS = 16
NW = NC * NS
BT = 4096
T = 21 * BT
NBT = T // BT
SC_ROWS = Q - T
ROWS_PER_W = SC_ROWS // NW
P = 457
NP = ROWS_PER_W // P
NSEARCH = 14
NEG_INF = float("-inf")


def _sc_segmax_body(q_hbm, pa_hbm, out_hbm, buf0, buf1, acc, pa_buf, bounds_smem, sem0, sem1):
    wid = lax.axis_index("s") * NC + lax.axis_index("c")
    lo = T + wid * ROWS_PER_W

    def start(p, buf, sem):
        pltpu.async_copy(q_hbm.at[pl.ds((lo + p * P) * D, P * D)], buf, sem)

    def wait(buf, sem):
        pltpu.make_async_copy(q_hbm.at[pl.ds(0, P * D)], buf, sem).wait()

    start(0, buf0, sem0)
    start(1, buf1, sem1)

    pltpu.sync_copy(pa_hbm.at[pl.ds(lo, ROWS_PER_W)], pa_buf.at[pl.ds(0, ROWS_PER_W)])

    def cbody(c, _):
        def sbody(_, lh):
            lo_i, hi_i = lh
            mid = (lo_i + hi_i) // 2
            v = pa_buf[pl.ds(mid, 16)]
            pred = v[0] < c
            return (
                jnp.where(pred, mid + 1, lo_i),
                jnp.where(pred, hi_i, mid),
            )

        l, _ = lax.fori_loop(
            0, NSEARCH, sbody, (jnp.int32(0), jnp.int32(ROWS_PER_W))
        )
        bounds_smem[c] = l
        return 0

    bounds_smem[0] = jnp.int32(0)
    bounds_smem[C] = jnp.int32(ROWS_PER_W)
    lax.fori_loop(1, C, cbody, 0)
    starts_s = [bounds_smem[c] for c in range(C + 1)]

    neg = jnp.full((16,), NEG_INF, jnp.float32)

    def init_body(i, _):
        acc[pl.ds(i * 16, 16)] = neg
        return 0

    lax.fori_loop(0, (CP * D) // 16, init_body, 0)

    def process(p, buf):
        row0 = p * P
        for c in range(C):
            rs = jnp.clip(starts_s[c] - row0, 0, P)
            re = jnp.clip(starts_s[c + 1] - row0, 0, P)
            a = tuple(acc[pl.ds(c * D + dc * 16, 16)] for dc in range(8))

            def rbody(r, a):
                base = r * D
                return tuple(
                    jnp.maximum(a[dc], buf[pl.ds(base + dc * 16, 16)])
                    for dc in range(8)
                )

            a = plsc.parallel_loop(rs, re, unroll=4, carry=a)(rbody)
            for dc in range(8):
                acc[pl.ds(c * D + dc * 16, 16)] = a[dc]

    def gbody(g, _):
        p0 = 2 * g
        wait(buf0, sem0)
        process(p0, buf0)

        @pl.when(p0 + 2 < NP)
        def _():
            start(p0 + 2, buf0, sem0)

        wait(buf1, sem1)
        process(p0 + 1, buf1)

        @pl.when(p0 + 3 < NP)
        def _():
            start(p0 + 3, buf1, sem1)

        return 0

    lax.fori_loop(0, NP // 2, gbody, 0)
    if NP % 2:
        wait(buf0, sem0)
        process(NP - 1, buf0)
    pltpu.sync_copy(acc, out_hbm.at[pl.ds(wid * CP * D, CP * D)])


_sc_segmax = functools.partial(
    pl.kernel,
    out_type=jax.ShapeDtypeStruct((NW * CP * D,), jnp.float32),
    mesh=plsc.VectorSubcoreMesh(core_axis_name="c", subcore_axis_name="s"),
    scratch_types=[
        pltpu.VMEM((P * D,), jnp.float32),
        pltpu.VMEM((P * D,), jnp.float32),
        pltpu.VMEM((CP * D,), jnp.float32),
        pltpu.VMEM((ROWS_PER_W + 16,), jnp.int32),
        pltpu.SMEM((16,), jnp.int32),
        pltpu.SemaphoreType.DMA,
        pltpu.SemaphoreType.DMA,
    ],
)(_sc_segmax_body)


def _tc_segmax_body(pa_ref, x_ref, out_ref):
    i = pl.program_id(0)
    base_rows = BT // 128

    @pl.when(i == 0)
    def _():
        out_ref[:] = jnp.full((CP, D), NEG_INF, jnp.float32)

    pa_blk = pa_ref[:]
    x = x_ref[:]
    first = pa_blk[0, 0]
    last = pa_blk[base_rows - 1, 127]

    @pl.when(first == last)
    def _():
        m = jnp.max(x, axis=0, keepdims=True)
        cur = out_ref[pl.ds(first, 1), :]
        out_ref[pl.ds(first, 1), :] = jnp.maximum(cur, m)

    @pl.when(first != last)
    def _():
        riota = lax.broadcasted_iota(jnp.int32, (BT, 1), 0)
        less = [jnp.int32(0)]
        for c in range(C):
            less.append(jnp.sum((pa_blk < (c + 1)).astype(jnp.int32)))
        for c in range(C):
            mask = (riota >= less[c]) & (riota < less[c + 1])
            m = jnp.max(jnp.where(mask, x, NEG_INF), axis=0, keepdims=True)
            cur = out_ref[pl.ds(c, 1), :]
            out_ref[pl.ds(c, 1), :] = jnp.maximum(cur, m)


_tc_segmax = pl.pallas_call(
    _tc_segmax_body,
    grid=(NBT,),
    in_specs=[
        pl.BlockSpec((BT // 128, 128), lambda i: (i, 0)),
        pl.BlockSpec((BT, D), lambda i: (i, 0)),
    ],
    out_specs=pl.BlockSpec((CP, D), lambda i: (0, 0)),
    out_shape=jax.ShapeDtypeStruct((CP, D), jnp.float32),
)


def _tc_gcn_body(part_ref, tcp_ref, pad_ref, w_ref, b_ref, cc_ref, out_ref):
    seg = tcp_ref[:]
    for i in range(NW):
        seg = jnp.maximum(seg, part_ref[i])
    core = jnp.where(seg > NEG_INF, seg, pad_ref[:])

    cc = cc_ref[:]
    rr = lax.broadcasted_iota(jnp.int32, (CP, CP), 0)
    cidx = lax.broadcasted_iota(jnp.int32, (CP, CP), 1)
    eye = jnp.where(rr == cidx, 1.0, 0.0).astype(jnp.float32)
    a_hat = cc + eye
    deg = jnp.sum(a_hat, axis=1, keepdims=True)
    dinv = jnp.where(deg > 0, lax.rsqrt(deg), 0.0)
    t = jnp.dot(a_hat, core * dinv, preferred_element_type=jnp.float32)
    h = jnp.dot(t * dinv, w_ref[:], preferred_element_type=jnp.float32)
    out_ref[:] = jnp.maximum(h + b_ref[:], 0.0)


_tc_gcn = pl.pallas_call(
    _tc_gcn_body,
    out_shape=jax.ShapeDtypeStruct((CP, D), jnp.float32),
)


def kernel(prev_assign, q_embeddings, padding_emb, W, b, core_con):
    pa = prev_assign.astype(jnp.int32)
    partials = _sc_segmax(q_embeddings.reshape(-1), pa)
    partials = partials.reshape(NW, CP, D)
    tc_part = _tc_segmax(pa.reshape(Q // 128, 128), q_embeddings)

    cc16 = jnp.zeros((CP, CP), jnp.float32).at[:C, :C].set(core_con)
    out16 = _tc_gcn(
        partials,
        tc_part,
        padding_emb.reshape(1, D),
        W,
        b.reshape(1, D),
        cc16,
    )
    return out16[:C]


# speedup vs baseline: 1.1842x; 1.0075x over previous
import functools

import jax
import jax.numpy as jnp
from jax import lax
from jax.experimental import pallas as pl
from jax.experimental.pallas import tpu as pltpu
from jax.experimental.pallas import tpu_sc as plsc

Q = 320000
D = 128
C = 10
CP = 16
NC = 2
NS = 16
NW = NC * NS
BT = 4096
T = 28 * BT
NBT = T // BT
SC_ROWS = Q - T
ROWS_PER_W = SC_ROWS // NW
P = 401
NP = ROWS_PER_W // P
NSEARCH = 14
NEG_INF = float("-inf")


def _sc_segmax_body(q_hbm, pa_hbm, out_hbm, buf0, buf1, acc, pa_buf, bounds_smem, sem0, sem1):
    wid = lax.axis_index("s") * NC + lax.axis_index("c")
    lo = T + wid * ROWS_PER_W

    def start(p, buf, sem):
        pltpu.async_copy(q_hbm.at[pl.ds((lo + p * P) * D, P * D)], buf, sem)

    def wait(buf, sem):
        pltpu.make_async_copy(q_hbm.at[pl.ds(0, P * D)], buf, sem).wait()

    start(0, buf0, sem0)
    start(1, buf1, sem1)

    pltpu.sync_copy(pa_hbm.at[pl.ds(lo, ROWS_PER_W)], pa_buf.at[pl.ds(0, ROWS_PER_W)])

    def cbody(c, _):
        def sbody(_, lh):
            lo_i, hi_i = lh
            mid = (lo_i + hi_i) // 2
            v = pa_buf[pl.ds(mid, 16)]
            pred = v[0] < c
            return (
                jnp.where(pred, mid + 1, lo_i),
                jnp.where(pred, hi_i, mid),
            )

        l, _ = lax.fori_loop(
            0, NSEARCH, sbody, (jnp.int32(0), jnp.int32(ROWS_PER_W))
        )
        bounds_smem[c] = l
        return 0

    bounds_smem[0] = jnp.int32(0)
    bounds_smem[C] = jnp.int32(ROWS_PER_W)
    lax.fori_loop(1, C, cbody, 0)
    starts_s = [bounds_smem[c] for c in range(C + 1)]

    neg = jnp.full((16,), NEG_INF, jnp.float32)

    def init_body(i, _):
        acc[pl.ds(i * 16, 16)] = neg
        return 0

    lax.fori_loop(0, (CP * D) // 16, init_body, 0)

    def process(p, buf):
        row0 = p * P
        for c in range(C):
            rs = jnp.clip(starts_s[c] - row0, 0, P)
            re = jnp.clip(starts_s[c + 1] - row0, 0, P)
            a = tuple(acc[pl.ds(c * D + dc * 16, 16)] for dc in range(8))

            def rbody(r, a):
                base = r * D
                return tuple(
                    jnp.maximum(a[dc], buf[pl.ds(base + dc * 16, 16)])
                    for dc in range(8)
                )

            a = plsc.parallel_loop(rs, re, unroll=4, carry=a)(rbody)
            for dc in range(8):
                acc[pl.ds(c * D + dc * 16, 16)] = a[dc]

    def gbody(g, _):
        p0 = 2 * g
        wait(buf0, sem0)
        process(p0, buf0)

        @pl.when(p0 + 2 < NP)
        def _():
            start(p0 + 2, buf0, sem0)

        wait(buf1, sem1)
        process(p0 + 1, buf1)

        @pl.when(p0 + 3 < NP)
        def _():
            start(p0 + 3, buf1, sem1)

        return 0

    lax.fori_loop(0, NP // 2, gbody, 0)
    if NP % 2:
        wait(buf0, sem0)
        process(NP - 1, buf0)
    pltpu.sync_copy(acc, out_hbm.at[pl.ds(wid * CP * D, CP * D)])


_sc_segmax = functools.partial(
    pl.kernel,
    out_type=jax.ShapeDtypeStruct((NW * CP * D,), jnp.float32),
    mesh=plsc.VectorSubcoreMesh(core_axis_name="c", subcore_axis_name="s"),
    scratch_types=[
        pltpu.VMEM((P * D,), jnp.float32),
        pltpu.VMEM((P * D,), jnp.float32),
        pltpu.VMEM((CP * D,), jnp.float32),
        pltpu.VMEM((ROWS_PER_W + 16,), jnp.int32),
        pltpu.SMEM((16,), jnp.int32),
        pltpu.SemaphoreType.DMA,
        pltpu.SemaphoreType.DMA,
    ],
)(_sc_segmax_body)


def _tc_segmax_body(pa_ref, x_ref, out_ref):
    i = pl.program_id(0)
    base_rows = BT // 128

    @pl.when(i == 0)
    def _():
        out_ref[:] = jnp.full((CP, D), NEG_INF, jnp.float32)

    pa_blk = pa_ref[:]
    x = x_ref[:]
    first = pa_blk[0, 0]
    last = pa_blk[base_rows - 1, 127]

    @pl.when(first == last)
    def _():
        m = jnp.max(x, axis=0, keepdims=True)
        cur = out_ref[pl.ds(first, 1), :]
        out_ref[pl.ds(first, 1), :] = jnp.maximum(cur, m)

    @pl.when(first != last)
    def _():
        riota = lax.broadcasted_iota(jnp.int32, (BT, 1), 0)
        less = [jnp.int32(0)]
        for c in range(C):
            less.append(jnp.sum((pa_blk < (c + 1)).astype(jnp.int32)))
        for c in range(C):
            mask = (riota >= less[c]) & (riota < less[c + 1])
            m = jnp.max(jnp.where(mask, x, NEG_INF), axis=0, keepdims=True)
            cur = out_ref[pl.ds(c, 1), :]
            out_ref[pl.ds(c, 1), :] = jnp.maximum(cur, m)


_tc_segmax = pl.pallas_call(
    _tc_segmax_body,
    grid=(NBT,),
    in_specs=[
        pl.BlockSpec((BT // 128, 128), lambda i: (i, 0)),
        pl.BlockSpec((BT, D), lambda i: (i, 0)),
    ],
    out_specs=pl.BlockSpec((CP, D), lambda i: (0, 0)),
    out_shape=jax.ShapeDtypeStruct((CP, D), jnp.float32),
)


def _tc_gcn_body(part_ref, tcp_ref, pad_ref, w_ref, b_ref, cc_ref, out_ref):
    seg = tcp_ref[:]
    for i in range(NW):
        seg = jnp.maximum(seg, part_ref[i])
    core = jnp.where(seg > NEG_INF, seg, pad_ref[:])

    cc = cc_ref[:]
    rr = lax.broadcasted_iota(jnp.int32, (CP, CP), 0)
    cidx = lax.broadcasted_iota(jnp.int32, (CP, CP), 1)
    eye = jnp.where(rr == cidx, 1.0, 0.0).astype(jnp.float32)
    a_hat = cc + eye
    deg = jnp.sum(a_hat, axis=1, keepdims=True)
    dinv = jnp.where(deg > 0, lax.rsqrt(deg), 0.0)
    t = jnp.dot(a_hat, core * dinv, preferred_element_type=jnp.float32)
    h = jnp.dot(t * dinv, w_ref[:], preferred_element_type=jnp.float32)
    out_ref[:] = jnp.maximum(h + b_ref[:], 0.0)


_tc_gcn = pl.pallas_call(
    _tc_gcn_body,
    out_shape=jax.ShapeDtypeStruct((CP, D), jnp.float32),
)


def kernel(prev_assign, q_embeddings, padding_emb, W, b, core_con):
    pa = prev_assign.astype(jnp.int32)
    partials = _sc_segmax(q_embeddings.reshape(-1), pa)
    partials = partials.reshape(NW, CP, D)
    tc_part = _tc_segmax(pa.reshape(Q // 128, 128), q_embeddings)

    cc16 = jnp.zeros((CP, CP), jnp.float32).at[:C, :C].set(core_con)
    out16 = _tc_gcn(
        partials,
        tc_part,
        padding_emb.reshape(1, D),
        W,
        b.reshape(1, D),
        cc16,
    )
    return out16[:C]
